# Initial kernel scaffold; baseline (speedup 1.0000x reference)
import jax
import jax.numpy as jnp
from jax.experimental import pallas as pl


def kernel(x, edge_index, Ws1, Wn1, b1, Ws2, Wn2, b2, Ws3, Wn3, b3):
    raise NotImplementedError("write your pallas kernel here")


# speedup vs baseline: 1.2002x; 1.2002x over previous
import functools

import jax
import jax.numpy as jnp
from jax import lax
from jax.experimental import pallas as pl
from jax.experimental.pallas import tpu as pltpu
from jax.experimental.pallas import tpu_sc as plsc

N = 10000
E = 320000
D = 128
H = 128
C = 64

NC = 2
NS = 16
NW = NC * NS
LANES = 16

CHUNK = 128
K = 80
KH = K // 2
EPW = K * CHUNK
NPAD = 10240
ROWS_PER_TILE = NPAD // NS
RB = 1000
DEG_DEPTH = 4


def _agg_body(h_hbm, src_hbm, dst_hbm, out_hbm,
              acc, src_v, dst_v, rows0, rows1, sem0, sem1):
    c = lax.axis_index("c")
    s = lax.axis_index("s")
    wid = s * NC + c

    zero16 = jnp.zeros((LANES,), jnp.float32)

    def _zrow(i, _):
        for l in range(H // LANES):
            rows0[i, pl.ds(l * LANES, LANES)] = zero16
        return 0
    lax.fori_loop(0, CHUNK, _zrow, 0)
    for t in range(ROWS_PER_TILE // CHUNK):
        pltpu.sync_copy(rows0, acc.at[pl.ds(s * ROWS_PER_TILE + t * CHUNK, CHUNK)])

    plsc.subcore_barrier()

    for ph in range(2):
        pltpu.sync_copy(src_hbm.at[wid, ph], src_v)
        pltpu.sync_copy(dst_hbm.at[wid, ph], dst_v)

        pltpu.async_copy(h_hbm.at[src_v.at[0]], rows0, sem0)
        pltpu.async_copy(h_hbm.at[src_v.at[1]], rows1, sem1)

        def _edge_chunk(j, rows, sem):
            pltpu.make_async_copy(h_hbm.at[src_v.at[0]], rows, sem).wait()
            pltpu.sync_copy(rows, acc.at[dst_v.at[j]], add=True)
            pltpu.async_copy(h_hbm.at[src_v.at[j + 2]], rows, sem)

        def _main(jj, _):
            _edge_chunk(2 * jj, rows0, sem0)
            _edge_chunk(2 * jj + 1, rows1, sem1)
            return 0
        lax.fori_loop(0, KH // 2, _main, 0)

        pltpu.make_async_copy(h_hbm.at[src_v.at[0]], rows0, sem0).wait()
        pltpu.make_async_copy(h_hbm.at[src_v.at[1]], rows1, sem1).wait()

    plsc.subcore_barrier()
    pltpu.sync_copy(acc.at[pl.ds(s * ROWS_PER_TILE, ROWS_PER_TILE)],
                    out_hbm.at[c, pl.ds(s * ROWS_PER_TILE, ROWS_PER_TILE)])


_sc_agg = pl.kernel(
    _agg_body,
    out_type=[jax.ShapeDtypeStruct((NC, NPAD, H), jnp.float32)],
    mesh=plsc.VectorSubcoreMesh(core_axis_name="c", subcore_axis_name="s"),
    scratch_types=[
        pltpu.VMEM_SHARED((NPAD, H), jnp.float32),
        pltpu.VMEM((KH + 2, CHUNK), jnp.int32),
        pltpu.VMEM((KH, CHUNK), jnp.int32),
        pltpu.VMEM((CHUNK, H), jnp.float32),
        pltpu.VMEM((CHUNK, H), jnp.float32),
        pltpu.SemaphoreType.DMA,
        pltpu.SemaphoreType.DMA,
    ],
)


def _layer_body(relu, x_ref, wsT_ref, b_ref, p0_ref, p1_ref,
                da_ref, db_ref, wnT_ref, o_ref):
    inv = 1.0 / jnp.maximum(da_ref[...] + db_ref[...], 1.0)
    agg = (p0_ref[...] + p1_ref[...]) * inv
    pre = (jnp.dot(x_ref[...], wsT_ref[...], preferred_element_type=jnp.float32)
           + jnp.dot(agg, wnT_ref[...], preferred_element_type=jnp.float32)
           + b_ref[...])
    o_ref[...] = jnp.maximum(pre, 0.0) if relu else pre


def _tc_layer(relu, x, wsT, b, p0, p1, da, db, wnT):
    hd = wsT.shape[1]
    return pl.pallas_call(
        functools.partial(_layer_body, relu),
        grid=(N // RB,),
        in_specs=[pl.BlockSpec((RB, H), lambda i: (i, 0)),
                  pl.BlockSpec((H, hd), lambda i: (0, 0)),
                  pl.BlockSpec((1, hd), lambda i: (0, 0)),
                  pl.BlockSpec((RB, H), lambda i: (i, 0)),
                  pl.BlockSpec((RB, H), lambda i: (i, 0)),
                  pl.BlockSpec((RB, 1), lambda i: (i, 0)),
                  pl.BlockSpec((RB, 1), lambda i: (i, 0)),
                  pl.BlockSpec((H, hd), lambda i: (0, 0))],
        out_specs=pl.BlockSpec((RB, hd), lambda i: (i, 0)),
        out_shape=jax.ShapeDtypeStruct((N, hd), jnp.float32),
    )(x, wsT, b, p0, p1, da, db, wnT)


def kernel(x, edge_index, Ws1, Wn1, b1, Ws2, Wn2, b2, Ws3, Wn3, b3):
    src = edge_index[0]
    dst = edge_index[1]

    pad = NW * EPW - E
    src_p = jnp.concatenate([src, jnp.zeros((pad,), jnp.int32)])
    dst_p = jnp.concatenate([dst, jnp.full((pad,), N, jnp.int32)])
    src_t = jnp.concatenate(
        [src_p.reshape(NW, 2, KH, CHUNK),
         jnp.zeros((NW, 2, 2, CHUNK), jnp.int32)], axis=2)
    dst_t = dst_p.reshape(NW, 2, KH, CHUNK)

    ws1T, wn1T = Ws1.T, Wn1.T
    ws2T, wn2T = Ws2.T, Wn2.T
    ws3T, wn3T = Ws3.T, Wn3.T
    b1r = b1.reshape(1, H)
    b2r = b2.reshape(1, H)
    b3r = b3.reshape(1, C)

    degp, = _sc_agg(jnp.ones((N, H), jnp.float32), src_t, dst_t)
    da = degp[0, :, 0:1]
    db = degp[1, :, 0:1]

    p1, = _sc_agg(x, src_t, dst_t)
    h1 = _tc_layer(True, x, ws1T, b1r, p1[0], p1[1], da, db, wn1T)
    p2, = _sc_agg(h1, src_t, dst_t)
    h2 = _tc_layer(True, h1, ws2T, b2r, p2[0], p2[1], da, db, wn2T)
    p3, = _sc_agg(h2, src_t, dst_t)
    out = _tc_layer(False, h2, ws3T, b3r, p3[0], p3[1], da, db, wn3T)
    return out


# speedup vs baseline: 1.5790x; 1.3156x over previous
import functools

import jax
import jax.numpy as jnp
from jax import lax
from jax.experimental import pallas as pl
from jax.experimental.pallas import tpu as pltpu
from jax.experimental.pallas import tpu_sc as plsc

N = 10000
E = 320000
D = 128
H = 128
C = 64

NC = 2
NS = 16
NW = NC * NS
LANES = 16

CHUNK = 128
K = 80
KH = K // 2
EPW = K * CHUNK
NPAD = 10240
ROWS_PER_TILE = NPAD // NS
RB = 1000
DEG_DEPTH = 4


def _agg_body(h_hbm, src_hbm, dst_hbm, out_hbm,
              acc, src_v, dst_v, rows0, rows1, sem0, sem1):
    c = lax.axis_index("c")
    s = lax.axis_index("s")
    wid = s * NC + c

    zero16 = jnp.zeros((LANES,), jnp.float32)

    def _zrow(i, _):
        for l in range(H // LANES):
            rows0[i, pl.ds(l * LANES, LANES)] = zero16
        return 0
    lax.fori_loop(0, CHUNK, _zrow, 0)
    for t in range(ROWS_PER_TILE // CHUNK):
        pltpu.sync_copy(rows0, acc.at[pl.ds(s * ROWS_PER_TILE + t * CHUNK, CHUNK)])

    plsc.subcore_barrier()

    for ph in range(2):
        pltpu.sync_copy(src_hbm.at[wid, ph], src_v)
        pltpu.sync_copy(dst_hbm.at[wid, ph], dst_v)

        pltpu.async_copy(h_hbm.at[src_v.at[0]], rows0, sem0)
        pltpu.async_copy(h_hbm.at[src_v.at[1]], rows1, sem1)

        def _edge_chunk(j, rows, sem):
            pltpu.make_async_copy(h_hbm.at[src_v.at[0]], rows, sem).wait()
            pltpu.sync_copy(rows, acc.at[dst_v.at[j]], add=True)
            pltpu.async_copy(h_hbm.at[src_v.at[j + 2]], rows, sem)

        def _main(jj, _):
            _edge_chunk(2 * jj, rows0, sem0)
            _edge_chunk(2 * jj + 1, rows1, sem1)
            return 0
        lax.fori_loop(0, KH // 2, _main, 0)

        pltpu.make_async_copy(h_hbm.at[src_v.at[0]], rows0, sem0).wait()
        pltpu.make_async_copy(h_hbm.at[src_v.at[1]], rows1, sem1).wait()

    plsc.subcore_barrier()
    pltpu.sync_copy(acc.at[pl.ds(s * ROWS_PER_TILE, ROWS_PER_TILE)],
                    out_hbm.at[c, pl.ds(s * ROWS_PER_TILE, ROWS_PER_TILE)])


_sc_agg = pl.kernel(
    _agg_body,
    out_type=[jax.ShapeDtypeStruct((NC, NPAD, H), jnp.float32)],
    mesh=plsc.VectorSubcoreMesh(core_axis_name="c", subcore_axis_name="s"),
    scratch_types=[
        pltpu.VMEM_SHARED((NPAD, H), jnp.float32),
        pltpu.VMEM((KH + 2, CHUNK), jnp.int32),
        pltpu.VMEM((KH, CHUNK), jnp.int32),
        pltpu.VMEM((CHUNK, H), jnp.float32),
        pltpu.VMEM((CHUNK, H), jnp.float32),
        pltpu.SemaphoreType.DMA,
        pltpu.SemaphoreType.DMA,
    ],
)


def _deg_body(dst_hbm, deg_hbm, degacc, dst_v, onesbuf, sem):
    c = lax.axis_index("c")
    s = lax.axis_index("s")
    wid = s * NC + c

    zero16 = jnp.zeros((LANES,), jnp.float32)
    ones16 = jnp.ones((LANES,), jnp.float32)

    def _zd(i, _):
        onesbuf[i, :] = zero16
        return 0
    lax.fori_loop(0, CHUNK, _zd, 0)
    for t in range(ROWS_PER_TILE // CHUNK):
        pltpu.sync_copy(
            onesbuf, degacc.at[pl.ds(s * ROWS_PER_TILE + t * CHUNK, CHUNK)])

    def _od(i, _):
        onesbuf[i, :] = ones16
        return 0
    lax.fori_loop(0, CHUNK, _od, 0)

    pltpu.sync_copy(dst_hbm.at[wid], dst_v)
    plsc.subcore_barrier()

    def _body(j, _):
        pltpu.sync_copy(onesbuf, degacc.at[dst_v.at[j]], add=True)
        return 0
    lax.fori_loop(0, K, _body, 0)

    plsc.subcore_barrier()
    pltpu.sync_copy(degacc.at[pl.ds(s * ROWS_PER_TILE, ROWS_PER_TILE)],
                    deg_hbm.at[c, pl.ds(s * ROWS_PER_TILE, ROWS_PER_TILE)])


_sc_deg = pl.kernel(
    _deg_body,
    out_type=[jax.ShapeDtypeStruct((NC, NPAD, LANES), jnp.float32)],
    mesh=plsc.VectorSubcoreMesh(core_axis_name="c", subcore_axis_name="s"),
    scratch_types=[
        pltpu.VMEM_SHARED((NPAD, LANES), jnp.float32),
        pltpu.VMEM((K, CHUNK), jnp.int32),
        pltpu.VMEM((CHUNK, LANES), jnp.float32),
        pltpu.SemaphoreType.DMA,
    ],
    compiler_params=pltpu.CompilerParams(use_tc_tiling_on_sc=False),
)


def _layer_body(relu, x_ref, wsT_ref, b_ref, p0_ref, p1_ref,
                da_ref, db_ref, wnT_ref, o_ref):
    inv = 1.0 / jnp.maximum(da_ref[...] + db_ref[...], 1.0)
    agg = (p0_ref[...] + p1_ref[...]) * inv
    pre = (jnp.dot(x_ref[...], wsT_ref[...], preferred_element_type=jnp.float32)
           + jnp.dot(agg, wnT_ref[...], preferred_element_type=jnp.float32)
           + b_ref[...])
    o_ref[...] = jnp.maximum(pre, 0.0) if relu else pre


def _tc_layer(relu, x, wsT, b, p0, p1, da, db, wnT):
    hd = wsT.shape[1]
    return pl.pallas_call(
        functools.partial(_layer_body, relu),
        grid=(N // RB,),
        in_specs=[pl.BlockSpec((RB, H), lambda i: (i, 0)),
                  pl.BlockSpec((H, hd), lambda i: (0, 0)),
                  pl.BlockSpec((1, hd), lambda i: (0, 0)),
                  pl.BlockSpec((RB, H), lambda i: (i, 0)),
                  pl.BlockSpec((RB, H), lambda i: (i, 0)),
                  pl.BlockSpec((RB, 1), lambda i: (i, 0)),
                  pl.BlockSpec((RB, 1), lambda i: (i, 0)),
                  pl.BlockSpec((H, hd), lambda i: (0, 0))],
        out_specs=pl.BlockSpec((RB, hd), lambda i: (i, 0)),
        out_shape=jax.ShapeDtypeStruct((N, hd), jnp.float32),
    )(x, wsT, b, p0, p1, da, db, wnT)


def kernel(x, edge_index, Ws1, Wn1, b1, Ws2, Wn2, b2, Ws3, Wn3, b3):
    src = edge_index[0]
    dst = edge_index[1]

    pad = NW * EPW - E
    src_p = jnp.concatenate([src, jnp.zeros((pad,), jnp.int32)])
    dst_p = jnp.concatenate([dst, jnp.full((pad,), N, jnp.int32)])
    src_t = jnp.concatenate(
        [src_p.reshape(NW, 2, KH, CHUNK),
         jnp.zeros((NW, 2, 2, CHUNK), jnp.int32)], axis=2)
    dst_t = dst_p.reshape(NW, 2, KH, CHUNK)
    dst_d = dst_p.reshape(NW, K, CHUNK)

    ws1T, wn1T = Ws1.T, Wn1.T
    ws2T, wn2T = Ws2.T, Wn2.T
    ws3T, wn3T = Ws3.T, Wn3.T
    b1r = b1.reshape(1, H)
    b2r = b2.reshape(1, H)
    b3r = b3.reshape(1, C)

    degp, = _sc_deg(dst_d)
    da = degp[0, :, 0:1]
    db = degp[1, :, 0:1]

    p1, = _sc_agg(x, src_t, dst_t)
    h1 = _tc_layer(True, x, ws1T, b1r, p1[0], p1[1], da, db, wn1T)
    p2, = _sc_agg(h1, src_t, dst_t)
    h2 = _tc_layer(True, h1, ws2T, b2r, p2[0], p2[1], da, db, wn2T)
    p3, = _sc_agg(h2, src_t, dst_t)
    out = _tc_layer(False, h2, ws3T, b3r, p3[0], p3[1], da, db, wn3T)
    return out


# speedup vs baseline: 1.5820x; 1.0019x over previous
import functools

import jax
import jax.numpy as jnp
from jax import lax
from jax.experimental import pallas as pl
from jax.experimental.pallas import tpu as pltpu
from jax.experimental.pallas import tpu_sc as plsc

N = 10000
E = 320000
D = 128
H = 128
C = 64

NC = 2
NS = 16
NW = NC * NS
LANES = 16

CHUNK = 128
K = 80
KH = K // 2
EPW = K * CHUNK
NPAD = 10240
ROWS_PER_TILE = NPAD // NS
RB = 1000
DEG_DEPTH = 4


def _agg_body(h_hbm, src_hbm, dst_hbm, out_hbm,
              acc, src_v, dst_v, rows0, rows1, sem0, sem1):
    c = lax.axis_index("c")
    s = lax.axis_index("s")
    wid = s * NC + c

    zero16 = jnp.zeros((LANES,), jnp.float32)

    def _zrow(i, _):
        for l in range(H // LANES):
            rows0[i, pl.ds(l * LANES, LANES)] = zero16
        return 0
    lax.fori_loop(0, CHUNK, _zrow, 0)
    for t in range(ROWS_PER_TILE // CHUNK):
        pltpu.sync_copy(rows0, acc.at[pl.ds(s * ROWS_PER_TILE + t * CHUNK, CHUNK)])

    plsc.subcore_barrier()

    for ph in range(2):
        pltpu.sync_copy(src_hbm.at[wid, ph], src_v)
        pltpu.sync_copy(dst_hbm.at[wid, ph], dst_v)

        pltpu.async_copy(h_hbm.at[src_v.at[0]], rows0, sem0)
        pltpu.async_copy(h_hbm.at[src_v.at[1]], rows1, sem1)

        def _edge_chunk(j, rows, sem):
            pltpu.make_async_copy(h_hbm.at[src_v.at[0]], rows, sem).wait()
            pltpu.sync_copy(rows, acc.at[dst_v.at[j]], add=True)
            pltpu.async_copy(h_hbm.at[src_v.at[j + 2]], rows, sem)

        def _main(jj, _):
            _edge_chunk(2 * jj, rows0, sem0)
            _edge_chunk(2 * jj + 1, rows1, sem1)
            return 0
        lax.fori_loop(0, KH // 2, _main, 0)

        pltpu.make_async_copy(h_hbm.at[src_v.at[0]], rows0, sem0).wait()
        pltpu.make_async_copy(h_hbm.at[src_v.at[1]], rows1, sem1).wait()

    plsc.subcore_barrier()
    pltpu.sync_copy(acc.at[pl.ds(s * ROWS_PER_TILE, ROWS_PER_TILE)],
                    out_hbm.at[c, pl.ds(s * ROWS_PER_TILE, ROWS_PER_TILE)])


_sc_agg = pl.kernel(
    _agg_body,
    out_type=[jax.ShapeDtypeStruct((NC, NPAD, H), jnp.float32)],
    mesh=plsc.VectorSubcoreMesh(core_axis_name="c", subcore_axis_name="s"),
    scratch_types=[
        pltpu.VMEM_SHARED((NPAD, H), jnp.float32),
        pltpu.VMEM((KH + 2, CHUNK), jnp.int32),
        pltpu.VMEM((KH, CHUNK), jnp.int32),
        pltpu.VMEM((CHUNK, H), jnp.float32),
        pltpu.VMEM((CHUNK, H), jnp.float32),
        pltpu.SemaphoreType.DMA,
        pltpu.SemaphoreType.DMA,
    ],
)


def _deg_body(dst_hbm, deg_hbm, degacc, dst_v, onesbuf, sem):
    c = lax.axis_index("c")
    s = lax.axis_index("s")
    wid = s * NC + c

    zero16 = jnp.zeros((LANES,), jnp.float32)
    ones16 = jnp.ones((LANES,), jnp.float32)

    def _zd(i, _):
        onesbuf[i, :] = zero16
        return 0
    lax.fori_loop(0, CHUNK, _zd, 0)
    for t in range(ROWS_PER_TILE // CHUNK):
        pltpu.sync_copy(
            onesbuf, degacc.at[pl.ds(s * ROWS_PER_TILE + t * CHUNK, CHUNK)])

    def _od(i, _):
        onesbuf[i, :] = ones16
        return 0
    lax.fori_loop(0, CHUNK, _od, 0)

    pltpu.sync_copy(dst_hbm.at[wid], dst_v)
    plsc.subcore_barrier()

    def _body(j, _):
        pltpu.sync_copy(onesbuf, degacc.at[dst_v.at[j]], add=True)
        return 0
    lax.fori_loop(0, K, _body, 0)

    plsc.subcore_barrier()
    pltpu.sync_copy(degacc.at[pl.ds(s * ROWS_PER_TILE, ROWS_PER_TILE)],
                    deg_hbm.at[c, pl.ds(s * ROWS_PER_TILE, ROWS_PER_TILE)])


_sc_deg = pl.kernel(
    _deg_body,
    out_type=[jax.ShapeDtypeStruct((NC, NPAD, LANES), jnp.float32)],
    mesh=plsc.VectorSubcoreMesh(core_axis_name="c", subcore_axis_name="s"),
    scratch_types=[
        pltpu.VMEM_SHARED((NPAD, LANES), jnp.float32),
        pltpu.VMEM((K, CHUNK), jnp.int32),
        pltpu.VMEM((CHUNK, LANES), jnp.float32),
        pltpu.SemaphoreType.DMA,
    ],
    compiler_params=pltpu.CompilerParams(use_tc_tiling_on_sc=False),
)


def _layer_body(relu, x_ref, wsT_ref, b_ref, p0_ref, p1_ref,
                da_ref, db_ref, wnT_ref, o_ref):
    inv = 1.0 / jnp.maximum(da_ref[...] + db_ref[...], 1.0)
    agg = (p0_ref[...] + p1_ref[...]) * inv
    pre = (jnp.dot(x_ref[...], wsT_ref[...], preferred_element_type=jnp.float32)
           + jnp.dot(agg, wnT_ref[...], preferred_element_type=jnp.float32)
           + b_ref[...])
    o_ref[...] = jnp.maximum(pre, 0.0) if relu else pre


def _tc_layer(relu, x, wsT, b, p0, p1, da, db, wnT):
    hd = wsT.shape[1]
    return pl.pallas_call(
        functools.partial(_layer_body, relu),
        grid=(N // RB,),
        in_specs=[pl.BlockSpec((RB, H), lambda i: (i, 0)),
                  pl.BlockSpec((H, hd), lambda i: (0, 0)),
                  pl.BlockSpec((1, hd), lambda i: (0, 0)),
                  pl.BlockSpec((RB, H), lambda i: (i, 0)),
                  pl.BlockSpec((RB, H), lambda i: (i, 0)),
                  pl.BlockSpec((RB, 1), lambda i: (i, 0)),
                  pl.BlockSpec((RB, 1), lambda i: (i, 0)),
                  pl.BlockSpec((H, hd), lambda i: (0, 0))],
        out_specs=pl.BlockSpec((RB, hd), lambda i: (i, 0)),
        out_shape=jax.ShapeDtypeStruct((N, hd), jnp.float32),
    )(x, wsT, b, p0, p1, da, db, wnT)


def kernel(x, edge_index, Ws1, Wn1, b1, Ws2, Wn2, b2, Ws3, Wn3, b3):
    src = edge_index[0]
    dst = edge_index[1]

    pad = NW * EPW - E
    src_p = jnp.concatenate([src, jnp.zeros((pad,), jnp.int32)])
    dum = N + (jnp.arange(pad, dtype=jnp.int32) % (NPAD - N))
    dst_p = jnp.concatenate([dst, dum])
    src_t = jnp.concatenate(
        [src_p.reshape(NW, 2, KH, CHUNK),
         jnp.zeros((NW, 2, 2, CHUNK), jnp.int32)], axis=2)
    dst_t = dst_p.reshape(NW, 2, KH, CHUNK)
    dst_d = dst_p.reshape(NW, K, CHUNK)

    ws1T, wn1T = Ws1.T, Wn1.T
    ws2T, wn2T = Ws2.T, Wn2.T
    ws3T, wn3T = Ws3.T, Wn3.T
    b1r = b1.reshape(1, H)
    b2r = b2.reshape(1, H)
    b3r = b3.reshape(1, C)

    degp, = _sc_deg(dst_d)
    da = degp[0, :, 0:1]
    db = degp[1, :, 0:1]

    p1, = _sc_agg(x, src_t, dst_t)
    h1 = _tc_layer(True, x, ws1T, b1r, p1[0], p1[1], da, db, wn1T)
    p2, = _sc_agg(h1, src_t, dst_t)
    h2 = _tc_layer(True, h1, ws2T, b2r, p2[0], p2[1], da, db, wn2T)
    p3, = _sc_agg(h2, src_t, dst_t)
    out = _tc_layer(False, h2, ws3T, b3r, p3[0], p3[1], da, db, wn3T)
    return out
